# baseline (speedup 1.0000x reference)
import functools

import jax
import jax.numpy as jnp
from jax import lax
from jax.experimental import pallas as pl
from jax.experimental.pallas import tpu as pltpu
from jax.experimental.pallas import tpu_sc as plsc

D_MODEL = 64
NUM_CORES = 2
NUM_SUBCORES = 16
NUM_WORKERS = NUM_CORES * NUM_SUBCORES
CHUNK = 128


@functools.lru_cache(maxsize=None)
def _make_lookup(n_l: int, n_b1: int):
  nchunk = n_l * n_b1 // NUM_WORKERS
  assert nchunk * NUM_WORKERS == n_l * n_b1 and nchunk % 2 == 0
  mesh = plsc.VectorSubcoreMesh(
      core_axis_name="c", subcore_axis_name="s",
      num_cores=NUM_CORES, num_subcores=NUM_SUBCORES)

  @functools.partial(
      pl.kernel,
      out_type=jax.ShapeDtypeStruct((n_l, 8, n_b1, 8, CHUNK), jnp.float32),
      mesh=mesh,
      compiler_params=pltpu.CompilerParams(use_tc_tiling_on_sc=False,
                                           needs_layout_passes=False),
      scratch_types=[
          pltpu.VMEM((nchunk, CHUNK), jnp.int32),
          pltpu.VMEM((CHUNK, D_MODEL), jnp.float32),
          pltpu.VMEM((CHUNK, D_MODEL), jnp.float32),
          pltpu.VMEM((8, 8, CHUNK), jnp.float32),
          pltpu.VMEM((8, 8, CHUNK), jnp.float32),
          pltpu.SemaphoreType.DMA,
          pltpu.SemaphoreType.DMA,
          pltpu.SemaphoreType.DMA,
          pltpu.SemaphoreType.DMA,
          pltpu.SemaphoreType.DMA,
      ],
  )
  def lookup(idx_hbm, table_hbm, out_hbm, idx_v, rows0, rows1, tr0, tr1,
             sem_i, sem_g0, sem_g1, sem_o0, sem_o1):
    wid = lax.axis_index("s") * NUM_CORES + lax.axis_index("c")
    rows = (rows0, rows1)
    trans = (tr0, tr1)
    sem_g = (sem_g0, sem_g1)
    sem_o = (sem_o0, sem_o1)
    lane = lax.broadcasted_iota(jnp.int32, (16,), 0)

    def chunk_lb(j):
      cid = wid * nchunk + j
      return cid // n_b1, cid % n_b1

    def start_gather(j, p):
      pltpu.async_copy(table_hbm.at[idx_v.at[j]], rows[p], sem_g[p])

    def wait_gather(j, p):
      pltpu.make_async_copy(table_hbm.at[idx_v.at[j]], rows[p],
                            sem_g[p]).wait()

    cgroups = []
    for k in range(D_MODEL // 16):
      c = lane + 16 * k
      cgroups.append((c // 8, c % 8))

    def transpose(p):
      def g_body(g, carry):
        base = g * 16
        for bb in range(16):
          b0 = base + bb
          b0v = jnp.full((16,), b0, jnp.int32)
          for k, (c8v, c0v) in enumerate(cgroups):
            vals = rows[p][b0, pl.ds(16 * k, 16)]
            plsc.store_scatter(trans[p], [c8v, c0v, b0v], vals)
        return carry
      lax.fori_loop(0, CHUNK // 16, g_body, None)

    def start_out(j, p):
      l, b1 = chunk_lb(j)
      pltpu.async_copy(trans[p], out_hbm.at[l, :, b1], sem_o[p])

    def wait_out(j, p):
      l, b1 = chunk_lb(j)
      pltpu.make_async_copy(trans[p], out_hbm.at[l, :, b1], sem_o[p]).wait()

    pltpu.async_copy(idx_hbm.at[wid], idx_v, sem_i).wait()

    start_gather(0, 0)
    start_gather(1, 1)
    wait_gather(0, 0)
    transpose(0)
    start_gather(2, 0)
    start_out(0, 0)
    wait_gather(1, 1)
    transpose(1)
    start_gather(3, 1)
    start_out(1, 1)

    def pair_body(jj, carry):
      for p in range(2):
        j = 2 * jj + p
        wait_gather(j, p)
        wait_out(j - 2, p)
        transpose(p)
        start_gather(j + 2, p)
        start_out(j, p)
      return carry

    lax.fori_loop(1, nchunk // 2 - 1, pair_body, None)

    for p in range(2):
      j = nchunk - 2 + p
      wait_gather(j, p)
      wait_out(j - 2, p)
      transpose(p)
      start_out(j, p)
    wait_out(nchunk - 2, 0)
    wait_out(nchunk - 1, 1)

  return lookup


def kernel(x, embedding_weight):
  b, l = x.shape
  d = embedding_weight.shape[1]
  assert d == D_MODEL
  n_b1 = b // CHUNK
  idx = jnp.transpose(x).reshape(NUM_WORKERS, (l * n_b1) // NUM_WORKERS,
                                 CHUNK).astype(jnp.int32)
  out5 = _make_lookup(l, n_b1)(idx, embedding_weight)
  return out5.transpose(2, 4, 0, 1, 3).reshape(b, l, d)


# speedup vs baseline: 1.5800x; 1.5800x over previous
import functools

import jax
import jax.numpy as jnp
from jax import lax
from jax.experimental import pallas as pl
from jax.experimental.pallas import tpu as pltpu
from jax.experimental.pallas import tpu_sc as plsc

D_MODEL = 64
NUM_CORES = 2
NUM_SUBCORES = 16
NUM_WORKERS = NUM_CORES * NUM_SUBCORES
CHUNK = 128


@functools.lru_cache(maxsize=None)
def _make_lookup(n_l: int, n_b1: int):
  nchunk = n_l * n_b1 // NUM_WORKERS
  assert nchunk * NUM_WORKERS == n_l * n_b1 and nchunk % 2 == 0
  mesh = plsc.VectorSubcoreMesh(
      core_axis_name="c", subcore_axis_name="s",
      num_cores=NUM_CORES, num_subcores=NUM_SUBCORES)

  @functools.partial(
      pl.kernel,
      out_type=jax.ShapeDtypeStruct((n_l, 8, n_b1, 8, CHUNK), jnp.float32),
      mesh=mesh,
      compiler_params=pltpu.CompilerParams(use_tc_tiling_on_sc=False,
                                           needs_layout_passes=False),
      scratch_types=[
          pltpu.VMEM((nchunk, CHUNK), jnp.int32),
          pltpu.VMEM((CHUNK, D_MODEL), jnp.float32),
          pltpu.VMEM((CHUNK, D_MODEL), jnp.float32),
          pltpu.VMEM((8, 8, CHUNK), jnp.float32),
          pltpu.VMEM((8, 8, CHUNK), jnp.float32),
          pltpu.SemaphoreType.DMA,
          pltpu.SemaphoreType.DMA,
          pltpu.SemaphoreType.DMA,
          pltpu.SemaphoreType.DMA,
          pltpu.SemaphoreType.DMA,
      ],
  )
  def lookup(idx_hbm, table_hbm, out_hbm, idx_v, rows0, rows1, tr0, tr1,
             sem_i, sem_g0, sem_g1, sem_o0, sem_o1):
    wid = lax.axis_index("s") * NUM_CORES + lax.axis_index("c")
    rows = (rows0, rows1)
    trans = (tr0, tr1)
    sem_g = (sem_g0, sem_g1)
    sem_o = (sem_o0, sem_o1)
    lane = lax.broadcasted_iota(jnp.int32, (16,), 0)

    def chunk_lb(j):
      cid = wid * nchunk + j
      return cid // n_b1, cid % n_b1

    def start_gather(j, p):
      pltpu.async_copy(table_hbm.at[idx_v.at[j]], rows[p], sem_g[p])

    def wait_gather(j, p):
      pltpu.make_async_copy(table_hbm.at[idx_v.at[j]], rows[p],
                            sem_g[p]).wait()

    cgroups = []
    for k in range(D_MODEL // 16):
      c = lane + 16 * k
      cgroups.append((c, c // 8, c % 8))

    def transpose(p):
      def g_body(g, carry):
        base = g * 16
        for r in range(16):
          b0v = base + ((lane + r) & 15)
          for cv, c8v, c0v in cgroups:
            vals = plsc.load_gather(rows[p], [b0v, cv])
            plsc.store_scatter(trans[p], [c8v, c0v, b0v], vals)
        return carry
      lax.fori_loop(0, CHUNK // 16, g_body, None)

    def start_out(j, p):
      l, b1 = chunk_lb(j)
      pltpu.async_copy(trans[p], out_hbm.at[l, :, b1], sem_o[p])

    def wait_out(j, p):
      l, b1 = chunk_lb(j)
      pltpu.make_async_copy(trans[p], out_hbm.at[l, :, b1], sem_o[p]).wait()

    pltpu.async_copy(idx_hbm.at[wid], idx_v, sem_i).wait()

    start_gather(0, 0)
    start_gather(1, 1)
    wait_gather(0, 0)
    transpose(0)
    start_gather(2, 0)
    start_out(0, 0)
    wait_gather(1, 1)
    transpose(1)
    start_gather(3, 1)
    start_out(1, 1)

    def pair_body(jj, carry):
      for p in range(2):
        j = 2 * jj + p
        wait_gather(j, p)
        wait_out(j - 2, p)
        transpose(p)
        start_gather(j + 2, p)
        start_out(j, p)
      return carry

    lax.fori_loop(1, nchunk // 2 - 1, pair_body, None)

    for p in range(2):
      j = nchunk - 2 + p
      wait_gather(j, p)
      wait_out(j - 2, p)
      transpose(p)
      start_out(j, p)
    wait_out(nchunk - 2, 0)
    wait_out(nchunk - 1, 1)

  return lookup


def kernel(x, embedding_weight):
  b, l = x.shape
  d = embedding_weight.shape[1]
  assert d == D_MODEL
  n_b1 = b // CHUNK
  idx = jnp.transpose(x).reshape(NUM_WORKERS, (l * n_b1) // NUM_WORKERS,
                                 CHUNK).astype(jnp.int32)
  out5 = _make_lookup(l, n_b1)(idx, embedding_weight)
  return out5.transpose(2, 4, 0, 1, 3).reshape(b, l, d)
